# speedup vs baseline: 5.0163x; 5.0163x over previous
import functools

import jax
import jax.numpy as jnp
from jax import lax
from jax.experimental import pallas as pl
from jax.experimental.pallas import tpu as pltpu
from jax.experimental.pallas import tpu_sc as plsc

B, H, V, D = 4096, 50, 100000, 128
NC, NS = 2, 16
NW = NC * NS
LANES = 16

SEG_PER_W = B // NW
SEG_CHUNK = 2
CHUNKS = SEG_PER_W // SEG_CHUNK
ROWS_PER_CHUNK = SEG_CHUNK * H
IDX_PER_W = SEG_PER_W * H


def _pool_body(idx_hbm, table_hbm, out_hbm, idx_v, buf_v, out_v, sem):
    wid = lax.axis_index("s") * NC + lax.axis_index("c")
    pltpu.sync_copy(idx_hbm.at[wid], idx_v)

    def chunk_body(g, _):
        cp = pltpu.async_copy(table_hbm.at[idx_v.at[g]], buf_v, sem)
        cp.wait()
        for s in range(SEG_CHUNK):
            accs = [jnp.zeros((LANES,), jnp.float32) for _ in range(D // LANES)]
            for r in range(H):
                row = s * H + r
                for c in range(D // LANES):
                    accs[c] = accs[c] + buf_v[row, pl.ds(c * LANES, LANES)]
            for c in range(D // LANES):
                out_v[g * SEG_CHUNK + s, pl.ds(c * LANES, LANES)] = (
                    accs[c] * (1.0 / H)
                )
        return 0

    lax.fori_loop(0, CHUNKS, chunk_body, 0)
    pltpu.sync_copy(out_v, out_hbm.at[wid])


@jax.jit
def _pool(click3, table):
    mesh = plsc.VectorSubcoreMesh(core_axis_name="c", subcore_axis_name="s")
    f = pl.kernel(
        _pool_body,
        out_type=jax.ShapeDtypeStruct((NW, SEG_PER_W, D), jnp.float32),
        mesh=mesh,
        scratch_types=[
            pltpu.VMEM((CHUNKS, ROWS_PER_CHUNK), jnp.int32),
            pltpu.VMEM((ROWS_PER_CHUNK, D), jnp.float32),
            pltpu.VMEM((SEG_PER_W, D), jnp.float32),
            pltpu.SemaphoreType.DMA,
        ],
    )
    return f(click3, table)


def _mlp_body(x_ref, w1_ref, b1_ref, w2_ref, b2_ref, w3_ref, b3_ref, o_ref):
    x = x_ref[...]
    h = jnp.dot(x, w1_ref[...], preferred_element_type=jnp.float32)
    h = jnp.maximum(h + b1_ref[...], 0.0)
    h = jnp.dot(h, w2_ref[...], preferred_element_type=jnp.float32)
    h = jnp.maximum(h + b2_ref[...], 0.0)
    h = jnp.dot(h, w3_ref[...], preferred_element_type=jnp.float32)
    o_ref[...] = jnp.maximum(h + b3_ref[...], 0.0)


def _mlp(feat, W1, b1, W2, b2, W3, b3):
    BLK = 512
    grid = (B // BLK,)
    full = lambda shape: pl.BlockSpec(shape, lambda i: (0, 0))
    return pl.pallas_call(
        _mlp_body,
        grid=grid,
        in_specs=[
            pl.BlockSpec((BLK, D), lambda i: (i, 0)),
            full(W1.shape),
            full((1, W1.shape[1])),
            full(W2.shape),
            full((1, W2.shape[1])),
            full(W3.shape),
            full((1, W3.shape[1])),
        ],
        out_specs=pl.BlockSpec((BLK, W3.shape[1]), lambda i: (i, 0)),
        out_shape=jax.ShapeDtypeStruct((B, W3.shape[1]), jnp.float32),
    )(feat, W1, b1.reshape(1, -1), W2, b2.reshape(1, -1), W3, b3.reshape(1, -1))


def kernel(click_history, table, W1, b1, W2, b2, W3, b3):
    click3 = click_history.astype(jnp.int32).reshape(NW, CHUNKS, ROWS_PER_CHUNK)
    feat = _pool(click3, table).reshape(B, D)
    return _mlp(feat, W1, b1, W2, b2, W3, b3)


# speedup vs baseline: 13.2106x; 2.6335x over previous
import functools

import jax
import jax.numpy as jnp
from jax import lax
from jax.experimental import pallas as pl
from jax.experimental.pallas import tpu as pltpu
from jax.experimental.pallas import tpu_sc as plsc

B, H, V, D = 4096, 50, 100000, 128
NC, NS = 2, 16
NW = NC * NS
LANES = 16

SEG_PER_W = B // NW
SEG_CHUNK = 2
CHUNKS = SEG_PER_W // SEG_CHUNK
ROWS_PER_CHUNK = SEG_CHUNK * H
IDX_PER_W = SEG_PER_W * H


NBUF = 4
R_UNROLL = 10


def _pool_body(idx_hbm, table_hbm, out_hbm, idx_v, b0, b1, b2, b3, out_v,
               s0, s1, s2, s3):
    wid = lax.axis_index("s") * NC + lax.axis_index("c")
    bufs = (b0, b1, b2, b3)
    sems = (s0, s1, s2, s3)
    pltpu.sync_copy(idx_hbm.at[wid], idx_v)

    for g in range(NBUF):
        pltpu.async_copy(table_hbm.at[idx_v.at[g]], bufs[g], sems[g])

    def chunk_group(i, _):
        for b in range(NBUF):
            g = i * NBUF + b
            buf = bufs[b]
            pltpu.make_async_copy(
                table_hbm.at[idx_v.at[g]], buf, sems[b]
            ).wait()
            for s in range(SEG_CHUNK):
                base = s * H

                def red(rb, accs, _base=base, _buf=buf):
                    for u in range(R_UNROLL):
                        row = _base + rb * R_UNROLL + u
                        accs = tuple(
                            accs[c] + _buf[row, pl.ds(c * LANES, LANES)]
                            for c in range(D // LANES)
                        )
                    return accs

                accs = lax.fori_loop(
                    0, H // R_UNROLL, red,
                    tuple(jnp.zeros((LANES,), jnp.float32)
                          for _ in range(D // LANES)),
                )
                for c in range(D // LANES):
                    out_v[g * SEG_CHUNK + s, pl.ds(c * LANES, LANES)] = (
                        accs[c] * (1.0 / H)
                    )

            @pl.when(g + NBUF < CHUNKS)
            def _fire():
                pltpu.async_copy(
                    table_hbm.at[idx_v.at[g + NBUF]], buf, sems[b]
                )

        return 0

    lax.fori_loop(0, CHUNKS // NBUF, chunk_group, 0)
    pltpu.sync_copy(out_v, out_hbm.at[wid])


@jax.jit
def _pool(click3, table):
    mesh = plsc.VectorSubcoreMesh(core_axis_name="c", subcore_axis_name="s")
    f = pl.kernel(
        _pool_body,
        out_type=jax.ShapeDtypeStruct((NW, SEG_PER_W, D), jnp.float32),
        mesh=mesh,
        scratch_types=[
            pltpu.VMEM((CHUNKS, ROWS_PER_CHUNK), jnp.int32),
            pltpu.VMEM((ROWS_PER_CHUNK, D), jnp.float32),
            pltpu.VMEM((ROWS_PER_CHUNK, D), jnp.float32),
            pltpu.VMEM((ROWS_PER_CHUNK, D), jnp.float32),
            pltpu.VMEM((ROWS_PER_CHUNK, D), jnp.float32),
            pltpu.VMEM((SEG_PER_W, D), jnp.float32),
            pltpu.SemaphoreType.DMA,
            pltpu.SemaphoreType.DMA,
            pltpu.SemaphoreType.DMA,
            pltpu.SemaphoreType.DMA,
        ],
    )
    return f(click3, table)


def _mlp_body(x_ref, w1_ref, b1_ref, w2_ref, b2_ref, w3_ref, b3_ref, o_ref):
    x = x_ref[...]
    h = jnp.dot(x, w1_ref[...], preferred_element_type=jnp.float32)
    h = jnp.maximum(h + b1_ref[...], 0.0)
    h = jnp.dot(h, w2_ref[...], preferred_element_type=jnp.float32)
    h = jnp.maximum(h + b2_ref[...], 0.0)
    h = jnp.dot(h, w3_ref[...], preferred_element_type=jnp.float32)
    o_ref[...] = jnp.maximum(h + b3_ref[...], 0.0)


def _mlp(feat, W1, b1, W2, b2, W3, b3):
    BLK = 512
    grid = (B // BLK,)
    full = lambda shape: pl.BlockSpec(shape, lambda i: (0, 0))
    return pl.pallas_call(
        _mlp_body,
        grid=grid,
        in_specs=[
            pl.BlockSpec((BLK, D), lambda i: (i, 0)),
            full(W1.shape),
            full((1, W1.shape[1])),
            full(W2.shape),
            full((1, W2.shape[1])),
            full(W3.shape),
            full((1, W3.shape[1])),
        ],
        out_specs=pl.BlockSpec((BLK, W3.shape[1]), lambda i: (i, 0)),
        out_shape=jax.ShapeDtypeStruct((B, W3.shape[1]), jnp.float32),
    )(feat, W1, b1.reshape(1, -1), W2, b2.reshape(1, -1), W3, b3.reshape(1, -1))


def kernel(click_history, table, W1, b1, W2, b2, W3, b3):
    click3 = click_history.astype(jnp.int32).reshape(NW, CHUNKS, ROWS_PER_CHUNK)
    feat = _pool(click3, table).reshape(B, D)
    return _mlp(feat, W1, b1, W2, b2, W3, b3)
